# Initial kernel scaffold; baseline (speedup 1.0000x reference)
import jax
import jax.numpy as jnp
from jax import lax
from jax.experimental import pallas as pl
from jax.experimental.pallas import tpu as pltpu
from jax.experimental.pallas import tpu_sc as plsc

B = 16384
L = 20
D_TITLE = 32
D_CAST = 16
D_KEY = 16
D_MOOD = 32
D_OUT = 96

_info = plsc.get_sparse_core_info()
NC, NS = _info.num_cores, _info.num_subcores
NW = NC * NS
B_PER_W = B // NW
CB = 128
NCH = B_PER_W // CB


def _sc_kernel(title_ids, cast_ids_flat, keyword_ids_flat, mood_ids,
               title_table, cast_table, key_table, mood_table,
               out_hbm,
               tidx_v, midx_v, cidx_v, kidx_v,
               t_rows, m_rows, c_rows, k_rows,
               out_v, sem):
  wid = lax.axis_index("s") * NC + lax.axis_index("c")

  def chunk_body(ch, _):
    row0 = wid * B_PER_W + ch * CB

    pltpu.sync_copy(title_ids.at[pl.ds(row0, CB)], tidx_v)
    pltpu.sync_copy(mood_ids.at[pl.ds(row0, CB)], midx_v)
    pltpu.sync_copy(cast_ids_flat.at[pl.ds(row0 * L, CB * L)], cidx_v)
    pltpu.sync_copy(keyword_ids_flat.at[pl.ds(row0 * L, CB * L)], kidx_v)

    cp_t = pltpu.make_async_copy(title_table.at[tidx_v], t_rows, sem)
    cp_m = pltpu.make_async_copy(mood_table.at[midx_v], m_rows, sem)
    cp_c = pltpu.make_async_copy(cast_table.at[cidx_v], c_rows, sem)
    cp_k = pltpu.make_async_copy(key_table.at[kidx_v], k_rows, sem)
    cp_t.start()
    cp_m.start()
    cp_c.start()
    cp_k.start()
    cp_t.wait()
    cp_m.wait()
    cp_c.wait()
    cp_k.wait()

    def row_body(i, _):
      out_v[i, pl.ds(0, 16)] = t_rows[i, pl.ds(0, 16)]
      out_v[i, pl.ds(16, 16)] = t_rows[i, pl.ds(16, 16)]
      base = i * L
      acc_c = c_rows[base, :]
      for j in range(1, L):
        acc_c = acc_c + c_rows[base + j, :]
      out_v[i, pl.ds(32, 16)] = acc_c * (1.0 / L)
      acc_k = k_rows[base, :]
      for j in range(1, L):
        acc_k = acc_k + k_rows[base + j, :]
      out_v[i, pl.ds(48, 16)] = acc_k * (1.0 / L)
      out_v[i, pl.ds(64, 16)] = m_rows[i, pl.ds(0, 16)]
      out_v[i, pl.ds(80, 16)] = m_rows[i, pl.ds(16, 16)]
      return 0

    lax.fori_loop(0, CB, row_body, 0)

    pltpu.sync_copy(out_v, out_hbm.at[pl.ds(row0, CB)])
    return 0

  lax.fori_loop(0, NCH, chunk_body, 0)


@jax.jit
def _run(title_ids, cast_ids_flat, keyword_ids_flat, mood_ids,
         title_table, cast_table, key_table, mood_table):
  mesh = plsc.VectorSubcoreMesh(core_axis_name="c", subcore_axis_name="s")
  return pl.kernel(
      _sc_kernel,
      mesh=mesh,
      out_type=jax.ShapeDtypeStruct((B, D_OUT), jnp.float32),
      scratch_types=[
          pltpu.VMEM((CB,), jnp.int32),
          pltpu.VMEM((CB,), jnp.int32),
          pltpu.VMEM((CB * L,), jnp.int32),
          pltpu.VMEM((CB * L,), jnp.int32),
          pltpu.VMEM((CB, D_TITLE), jnp.float32),
          pltpu.VMEM((CB, D_MOOD), jnp.float32),
          pltpu.VMEM((CB * L, D_CAST), jnp.float32),
          pltpu.VMEM((CB * L, D_KEY), jnp.float32),
          pltpu.VMEM((CB, D_OUT), jnp.float32),
          pltpu.SemaphoreType.DMA,
      ],
  )(title_ids, cast_ids_flat, keyword_ids_flat, mood_ids,
    title_table, cast_table, key_table, mood_table)


def kernel(title_ids, cast_ids, keyword_ids, mood_ids,
           title_table, cast_table, key_table, mood_table):
  cast_flat = cast_ids.reshape(-1)
  key_flat = keyword_ids.reshape(-1)
  return _run(title_ids, cast_flat, key_flat, mood_ids,
              title_table, cast_table, key_table, mood_table)


# speedup vs baseline: 11.6361x; 11.6361x over previous
import jax
import jax.numpy as jnp
from jax import lax
from jax.experimental import pallas as pl
from jax.experimental.pallas import tpu as pltpu
from jax.experimental.pallas import tpu_sc as plsc

B = 16384
L = 20
D_TITLE = 32
D_CAST = 16
D_KEY = 16
D_MOOD = 32
D_OUT = 96

_info = plsc.get_sparse_core_info()
NC, NS = _info.num_cores, _info.num_subcores
NW = NC * NS
B_PER_W = B // NW
CB = 128
NCH = B_PER_W // CB


def _sc_kernel(title_ids, cast_ids_flat, keyword_ids_flat, mood_ids,
               title_table, cast_table, key_table, mood_table,
               out_hbm,
               tidx_v, midx_v, cidx_v, kidx_v,
               t_rows, m_rows, c_rows, k_rows,
               out_v, sem):
  wid = lax.axis_index("s") * NC + lax.axis_index("c")

  def chunk_body(ch, _):
    row0 = wid * B_PER_W + ch * CB

    pltpu.sync_copy(title_ids.at[pl.ds(row0, CB)], tidx_v)
    pltpu.sync_copy(mood_ids.at[pl.ds(row0, CB)], midx_v)
    pltpu.sync_copy(cast_ids_flat.at[pl.ds(row0 * L, CB * L)], cidx_v)
    pltpu.sync_copy(keyword_ids_flat.at[pl.ds(row0 * L, CB * L)], kidx_v)

    cp_t = pltpu.make_async_copy(title_table.at[tidx_v], t_rows, sem)
    cp_m = pltpu.make_async_copy(mood_table.at[midx_v], m_rows, sem)
    cp_c = pltpu.make_async_copy(cast_table.at[cidx_v], c_rows, sem)
    cp_k = pltpu.make_async_copy(key_table.at[kidx_v], k_rows, sem)
    cp_t.start()
    cp_m.start()
    cp_c.start()
    cp_k.start()
    cp_t.wait()
    cp_m.wait()
    cp_c.wait()
    cp_k.wait()

    def row_body(i, _):
      out_v[i, pl.ds(0, 16)] = t_rows[i, pl.ds(0, 16)]
      out_v[i, pl.ds(16, 16)] = t_rows[i, pl.ds(16, 16)]
      base = i * L
      acc_c = c_rows[base, :]
      for j in range(1, L):
        acc_c = acc_c + c_rows[base + j, :]
      out_v[i, pl.ds(32, 16)] = acc_c * (1.0 / L)
      acc_k = k_rows[base, :]
      for j in range(1, L):
        acc_k = acc_k + k_rows[base + j, :]
      out_v[i, pl.ds(48, 16)] = acc_k * (1.0 / L)
      out_v[i, pl.ds(64, 16)] = m_rows[i, pl.ds(0, 16)]
      out_v[i, pl.ds(80, 16)] = m_rows[i, pl.ds(16, 16)]
      return 0

    lax.fori_loop(0, CB, row_body, 0)

    pltpu.sync_copy(out_v, out_hbm.at[pl.ds(row0, CB)])
    return 0

  lax.fori_loop(0, NCH, chunk_body, 0)


@jax.jit
def _run(title_ids, cast_ids_flat, keyword_ids_flat, mood_ids,
         title_table, cast_table, key_table, mood_table):
  mesh = plsc.VectorSubcoreMesh(core_axis_name="c", subcore_axis_name="s")
  return pl.kernel(
      _sc_kernel,
      mesh=mesh,
      compiler_params=pltpu.CompilerParams(use_tc_tiling_on_sc=False),
      out_type=jax.ShapeDtypeStruct((B, D_OUT), jnp.float32),
      scratch_types=[
          pltpu.VMEM((CB,), jnp.int32),
          pltpu.VMEM((CB,), jnp.int32),
          pltpu.VMEM((CB * L,), jnp.int32),
          pltpu.VMEM((CB * L,), jnp.int32),
          pltpu.VMEM((CB, D_TITLE), jnp.float32),
          pltpu.VMEM((CB, D_MOOD), jnp.float32),
          pltpu.VMEM((CB * L, D_CAST), jnp.float32),
          pltpu.VMEM((CB * L, D_KEY), jnp.float32),
          pltpu.VMEM((CB, D_OUT), jnp.float32),
          pltpu.SemaphoreType.DMA,
      ],
  )(title_ids, cast_ids_flat, keyword_ids_flat, mood_ids,
    title_table, cast_table, key_table, mood_table)


def kernel(title_ids, cast_ids, keyword_ids, mood_ids,
           title_table, cast_table, key_table, mood_table):
  cast_flat = cast_ids.reshape(-1)
  key_flat = keyword_ids.reshape(-1)
  return _run(title_ids, cast_flat, key_flat, mood_ids,
              title_table, cast_table, key_table, mood_table)


# speedup vs baseline: 15.3124x; 1.3159x over previous
import jax
import jax.numpy as jnp
from jax import lax
from jax.experimental import pallas as pl
from jax.experimental.pallas import tpu as pltpu
from jax.experimental.pallas import tpu_sc as plsc

B = 16384
L = 20
D_TITLE = 32
D_CAST = 16
D_KEY = 16
D_MOOD = 32
D_OUT = 96

_info = plsc.get_sparse_core_info()
NC, NS = _info.num_cores, _info.num_subcores
NW = NC * NS
BW = B // NW


def _sc_kernel(title_ids, castT, keyT, mood_ids,
               title_table, cast_table, key_table, mood_table,
               out_hbm,
               tidx_v, midx_v, cidx_v, kidx_v,
               t_rows, m_rows, acc_c, acc_k,
               sem_i, sem_g, sem_o):
  wid = lax.axis_index("s") * NC + lax.axis_index("c")
  row0 = wid * BW

  idx_cps = [
      pltpu.make_async_copy(title_ids.at[pl.ds(row0, BW)], tidx_v, sem_i),
      pltpu.make_async_copy(mood_ids.at[pl.ds(row0, BW)], midx_v, sem_i),
      pltpu.make_async_copy(castT.at[:, pl.ds(row0, BW)], cidx_v, sem_i),
      pltpu.make_async_copy(keyT.at[:, pl.ds(row0, BW)], kidx_v, sem_i),
  ]
  for cp in idx_cps:
    cp.start()

  def zero_body(i, _):
    acc_c[i, :] = jnp.zeros((16,), jnp.float32)
    acc_k[i, :] = jnp.zeros((16,), jnp.float32)
    return 0
  lax.fori_loop(0, BW, zero_body, 0)

  for cp in idx_cps:
    cp.wait()

  g_cps = [
      pltpu.async_copy(title_table.at[tidx_v], t_rows, sem_g),
      pltpu.async_copy(mood_table.at[midx_v], m_rows, sem_g),
  ]
  for j in range(L):
    g_cps.append(
        pltpu.async_copy(cast_table.at[cidx_v.at[j]], acc_c, sem_g, add=True))
    g_cps.append(
        pltpu.async_copy(key_table.at[kidx_v.at[j]], acc_k, sem_g, add=True))
  for cp in g_cps:
    cp.wait()

  inv_l = jnp.float32(1.0 / L)
  def scale_body(i, _):
    acc_c[i, :] = acc_c[i, :] * inv_l
    acc_k[i, :] = acc_k[i, :] * inv_l
    return 0
  lax.fori_loop(0, BW, scale_body, 0)

  rows = pl.ds(row0, BW)
  out_cps = [
      pltpu.make_async_copy(t_rows, out_hbm.at[rows, pl.ds(0, D_TITLE)], sem_o),
      pltpu.make_async_copy(acc_c, out_hbm.at[rows, pl.ds(32, D_CAST)], sem_o),
      pltpu.make_async_copy(acc_k, out_hbm.at[rows, pl.ds(48, D_KEY)], sem_o),
      pltpu.make_async_copy(m_rows, out_hbm.at[rows, pl.ds(64, D_MOOD)], sem_o),
  ]
  for cp in out_cps:
    cp.start()
  for cp in out_cps:
    cp.wait()


@jax.jit
def _run(title_ids, castT, keyT, mood_ids,
         title_table, cast_table, key_table, mood_table):
  mesh = plsc.VectorSubcoreMesh(core_axis_name="c", subcore_axis_name="s")
  return pl.kernel(
      _sc_kernel,
      mesh=mesh,
      compiler_params=pltpu.CompilerParams(use_tc_tiling_on_sc=False),
      out_type=jax.ShapeDtypeStruct((B, D_OUT), jnp.float32),
      scratch_types=[
          pltpu.VMEM((BW,), jnp.int32),
          pltpu.VMEM((BW,), jnp.int32),
          pltpu.VMEM((L, BW), jnp.int32),
          pltpu.VMEM((L, BW), jnp.int32),
          pltpu.VMEM((BW, D_TITLE), jnp.float32),
          pltpu.VMEM((BW, D_MOOD), jnp.float32),
          pltpu.VMEM((BW, D_CAST), jnp.float32),
          pltpu.VMEM((BW, D_KEY), jnp.float32),
          pltpu.SemaphoreType.DMA,
          pltpu.SemaphoreType.DMA,
          pltpu.SemaphoreType.DMA,
      ],
  )(title_ids, castT, keyT, mood_ids,
    title_table, cast_table, key_table, mood_table)


def kernel(title_ids, cast_ids, keyword_ids, mood_ids,
           title_table, cast_table, key_table, mood_table):
  return _run(title_ids, cast_ids.T, keyword_ids.T, mood_ids,
              title_table, cast_table, key_table, mood_table)
